# baseline (speedup 1.0000x reference)
import functools
import math

import jax
import jax.numpy as jnp
from jax import lax
from jax.experimental import pallas as pl
from jax.experimental.pallas import tpu as pltpu
from jax.experimental.pallas import tpu_sc as plsc

N = 10000
E = 320000
D = 128
BN_EPS = 1e-5

NC = 2
NS = 16
NW = NC * NS

N_PAD = 10240
CHUNK = 64
NCHUNK = 162
E_PAD = NW * CHUNK * NCHUNK
ROWS_PER_TILE = N_PAD // NS

BM = 1280
GRID = N_PAD // BM


def _make_propagate(ones_mode=False):
  mesh = plsc.VectorSubcoreMesh(core_axis_name="c", subcore_axis_name="s",
                                num_cores=NC, num_subcores=NS)

  @functools.partial(
      pl.kernel,
      out_type=jax.ShapeDtypeStruct((NC, N_PAD, D), jnp.float32),
      mesh=mesh,
      scratch_types=[
          [pltpu.VMEM((2, CHUNK), jnp.int32) for _ in range(6)],
          [pltpu.VMEM((CHUNK, D), jnp.float32) for _ in range(3)],
          pltpu.VMEM_SHARED((N_PAD, D), jnp.float32),
          pltpu.SemaphoreType.DMA,
          [pltpu.SemaphoreType.DMA for _ in range(3)],
          pltpu.SemaphoreType.DMA,
      ],
  )
  def prop(g_hbm, edges_hbm, out_hbm, ib, rows, acc, gsem, ssem, isem):
    cid = lax.axis_index("c")
    sid = lax.axis_index("s")
    tid = cid * NS + sid
    cb = tid * NCHUNK

    pltpu.async_copy(edges_hbm.at[cb], ib[0], isem)
    pltpu.async_copy(edges_hbm.at[cb + 1], ib[1], isem)

    fill16 = (jnp.ones if ones_mode else jnp.zeros)((16,), jnp.float32)
    zero16 = jnp.zeros((16,), jnp.float32)

    def zbody(r, c):
      for col in range(D // 16):
        rows[0][r, pl.ds(col * 16, 16)] = zero16
      return c

    def fbody(r, c):
      for col in range(D // 16):
        rows[0][r, pl.ds(col * 16, 16)] = fill16
        rows[1][r, pl.ds(col * 16, 16)] = fill16
        rows[2][r, pl.ds(col * 16, 16)] = fill16
      return c

    with jax.named_scope("acc_zero"):
      lax.fori_loop(0, CHUNK, zbody, 0)
      for k in range(ROWS_PER_TILE // CHUNK):
        pltpu.sync_copy(rows[0],
                        acc.at[pl.ds(sid * ROWS_PER_TILE + k * CHUNK, CHUNK)])
      if ones_mode:
        lax.fori_loop(0, CHUNK, fbody, 0)
      pltpu.make_async_copy(edges_hbm.at[cb], ib[0], isem).wait()
      if not ones_mode:
        pltpu.async_copy(g_hbm.at[ib[0].at[0]], rows[0], gsem)
      plsc.subcore_barrier()

    def body(it, carry):
      for q in range(6):
        j = it * 6 + q
        r = q % 3
        r1 = (q + 1) % 3
        if not ones_mode:
          pltpu.make_async_copy(g_hbm.at[ib[q].at[0]], rows[r], gsem).wait()
        pltpu.async_copy(rows[r], acc.at[ib[q].at[1]], ssem[r], add=True)

        @pl.when(j + 1 < NCHUNK)
        def _():
          pltpu.make_async_copy(edges_hbm.at[cb + j + 1], ib[(q + 1) % 6],
                                isem).wait()

        @pl.when(j >= 2)
        def _():
          pltpu.make_async_copy(rows[r1], acc.at[ib[q].at[1]], ssem[r1]).wait()

        if not ones_mode:
          @pl.when(j + 1 < NCHUNK)
          def _():
            pltpu.async_copy(g_hbm.at[ib[(q + 1) % 6].at[0]], rows[r1], gsem)

        @pl.when(j + 2 < NCHUNK)
        def _():
          pltpu.async_copy(edges_hbm.at[cb + j + 2], ib[(q + 2) % 6], isem)
      return carry

    with jax.named_scope("edge_loop"):
      lax.fori_loop(0, NCHUNK // 6, body, 0)
      pltpu.make_async_copy(rows[1], acc.at[ib[0].at[1]], ssem[1]).wait()
      pltpu.make_async_copy(rows[2], acc.at[ib[0].at[1]], ssem[2]).wait()
    with jax.named_scope("post_barrier"):
      plsc.subcore_barrier()

    with jax.named_scope("writeout"):
      pltpu.sync_copy(acc.at[pl.ds(sid * ROWS_PER_TILE, ROWS_PER_TILE)],
                      out_hbm.at[cid, pl.ds(sid * ROWS_PER_TILE, ROWS_PER_TILE)])

  return prop


@functools.cache
def _get_propagate(ones_mode=False):
  return _make_propagate(ones_mode)


def _propagate(g, edges):
  return _get_propagate()(g, edges)


def _degree(g, edges):
  return _get_propagate(True)(g, edges)


def _prep_dinv(deg_partials):
  def body(p_ref, o_ref):
    i = pl.program_id(0)
    deg = 1.0 + p_ref[0] + p_ref[1]
    dinv = lax.rsqrt(deg)
    row = i * BM + lax.broadcasted_iota(jnp.int32, (BM, D), 0)
    o_ref[...] = jnp.where(row < N, dinv, 0.0)

  return pl.pallas_call(
      body,
      grid=(GRID,),
      in_specs=[pl.BlockSpec((NC, BM, D), lambda i: (0, i, 0))],
      out_specs=pl.BlockSpec((BM, D), lambda i: (i, 0)),
      out_shape=jax.ShapeDtypeStruct((N_PAD, D), jnp.float32),
  )(deg_partials)


def _matmul_scale(x, W, dinv_b):
  def body(x_ref, w_ref, d_ref, o_ref):
    o_ref[...] = d_ref[...] * jnp.dot(x_ref[...], w_ref[...],
                                      preferred_element_type=jnp.float32)

  return pl.pallas_call(
      body,
      grid=(GRID,),
      in_specs=[
          pl.BlockSpec((BM, D), lambda i: (i, 0)),
          pl.BlockSpec((D, D), lambda i: (0, 0)),
          pl.BlockSpec((BM, D), lambda i: (i, 0)),
      ],
      out_specs=pl.BlockSpec((BM, D), lambda i: (i, 0)),
      out_shape=jax.ShapeDtypeStruct((N_PAD, D), jnp.float32),
  )(x, W, dinv_b)


def _combine_matmul(P, g, dinv_b, bvec, gamma, beta, W):
  bn_c = float(1.0 / math.sqrt(1.0 + BN_EPS))

  def body(p_ref, g_ref, d_ref, b_ref, ga_ref, be_ref, w_ref, o_ref):
    z = d_ref[...] * (p_ref[0] + p_ref[1] + g_ref[...]) + b_ref[...]
    z = jnp.maximum(z, 0.0) * (ga_ref[...] * bn_c) + be_ref[...]
    o_ref[...] = d_ref[...] * jnp.dot(z, w_ref[...],
                                      preferred_element_type=jnp.float32)

  return pl.pallas_call(
      body,
      grid=(GRID,),
      in_specs=[
          pl.BlockSpec((NC, BM, D), lambda i: (0, i, 0)),
          pl.BlockSpec((BM, D), lambda i: (i, 0)),
          pl.BlockSpec((BM, D), lambda i: (i, 0)),
          pl.BlockSpec((1, D), lambda i: (0, 0)),
          pl.BlockSpec((1, D), lambda i: (0, 0)),
          pl.BlockSpec((1, D), lambda i: (0, 0)),
          pl.BlockSpec((D, D), lambda i: (0, 0)),
      ],
      out_specs=pl.BlockSpec((BM, D), lambda i: (i, 0)),
      out_shape=jax.ShapeDtypeStruct((N_PAD, D), jnp.float32),
  )(P, g, dinv_b, bvec, gamma, beta, W)


def _final_mean(P, g, dinv_b, bvec):
  def body(p_ref, g_ref, d_ref, b_ref, o_ref):
    i = pl.program_id(0)
    z = d_ref[...] * (p_ref[0] + p_ref[1] + g_ref[...])
    row = i * BM + lax.broadcasted_iota(jnp.int32, (BM, D), 0)
    z = jnp.where(row < N, z, 0.0)
    part = jnp.sum(z, axis=0, keepdims=True)

    @pl.when(i == 0)
    def _():
      o_ref[...] = jnp.zeros_like(o_ref)

    o_ref[...] += part

    @pl.when(i == GRID - 1)
    def _():
      o_ref[...] = o_ref[...] * (1.0 / N) + b_ref[...]

  return pl.pallas_call(
      body,
      grid=(GRID,),
      in_specs=[
          pl.BlockSpec((NC, BM, D), lambda i: (0, i, 0)),
          pl.BlockSpec((BM, D), lambda i: (i, 0)),
          pl.BlockSpec((BM, D), lambda i: (i, 0)),
          pl.BlockSpec((1, D), lambda i: (0, 0)),
      ],
      out_specs=pl.BlockSpec((1, D), lambda i: (0, 0)),
      out_shape=jax.ShapeDtypeStruct((1, D), jnp.float32),
  )(P, g, dinv_b, bvec)


@jax.jit
def _run(x, edge_index, W1, b1, W2, b2, W3, b3, W4, b4,
         gamma1, beta1, gamma2, beta2, gamma3, beta3):
  src = edge_index[0]
  dst = edge_index[1]
  pad_e = N + (jnp.arange(E_PAD - E, dtype=jnp.int32) % (N_PAD - N))
  srcp = jnp.concatenate([src, pad_e])
  dstp = jnp.concatenate([dst, pad_e])
  edges = jnp.stack([srcp.reshape(-1, CHUNK), dstp.reshape(-1, CHUNK)], axis=1)
  x_pad = jnp.pad(x, ((0, N_PAD - N), (0, 0)))


  b1r = b1.reshape(1, D)
  b2r = b2.reshape(1, D)
  b3r = b3.reshape(1, D)
  b4r = b4.reshape(1, D)
  g1r = gamma1.reshape(1, D)
  g2r = gamma2.reshape(1, D)
  g3r = gamma3.reshape(1, D)
  be1r = beta1.reshape(1, D)
  be2r = beta2.reshape(1, D)
  be3r = beta3.reshape(1, D)

  deg_p = _degree(x_pad, edges)
  dinv_b = _prep_dinv(deg_p)

  g1 = _matmul_scale(x_pad, W1, dinv_b)
  P1 = _propagate(g1, edges)
  g2 = _combine_matmul(P1, g1, dinv_b, b1r, g1r, be1r, W2)
  P2 = _propagate(g2, edges)
  g3 = _combine_matmul(P2, g2, dinv_b, b2r, g2r, be2r, W3)
  P3 = _propagate(g3, edges)
  g4 = _combine_matmul(P3, g3, dinv_b, b3r, g3r, be3r, W4)
  P4 = _propagate(g4, edges)
  return _final_mean(P4, g4, dinv_b, b4r)


def kernel(x, edge_index, W1, b1, W2, b2, W3, b3, W4, b4,
           gamma1, beta1, gamma2, beta2, gamma3, beta3):
  return _run(x, edge_index, W1, b1, W2, b2, W3, b3, W4, b4,
              gamma1, beta1, gamma2, beta2, gamma3, beta3)


# speedup vs baseline: 1.2172x; 1.2172x over previous
import functools
import math

import jax
import jax.numpy as jnp
from jax import lax
from jax.experimental import pallas as pl
from jax.experimental.pallas import tpu as pltpu
from jax.experimental.pallas import tpu_sc as plsc

N = 10000
E = 320000
D = 128
BN_EPS = 1e-5

NC = 2
NS = 16
NW = NC * NS

N_PAD = 10240
CHUNK = 96
NCHUNK = 108
E_PAD = NW * CHUNK * NCHUNK
ROWS_PER_TILE = N_PAD // NS

BM = 1280
GRID = N_PAD // BM


def _make_propagate(ones_mode=False):
  mesh = plsc.VectorSubcoreMesh(core_axis_name="c", subcore_axis_name="s",
                                num_cores=NC, num_subcores=NS)

  @functools.partial(
      pl.kernel,
      out_type=jax.ShapeDtypeStruct((NC, N_PAD, D), jnp.float32),
      mesh=mesh,
      scratch_types=[
          [pltpu.VMEM((2, CHUNK), jnp.int32) for _ in range(4)],
          [pltpu.VMEM((CHUNK, D), jnp.float32) for _ in range(2)],
          pltpu.VMEM_SHARED((N_PAD, D), jnp.float32),
          pltpu.SemaphoreType.DMA,
          [pltpu.SemaphoreType.DMA for _ in range(2)],
          pltpu.SemaphoreType.DMA,
      ],
  )
  def prop(g_hbm, edges_hbm, out_hbm, ib, rows, acc, gsem, ssem, isem):
    cid = lax.axis_index("c")
    sid = lax.axis_index("s")
    tid = cid * NS + sid
    cb = tid * NCHUNK

    pltpu.async_copy(edges_hbm.at[cb], ib[0], isem)

    fill16 = (jnp.ones if ones_mode else jnp.zeros)((16,), jnp.float32)
    zero16 = jnp.zeros((16,), jnp.float32)

    def zbody(r, c):
      for col in range(D // 16):
        rows[0][r, pl.ds(col * 16, 16)] = zero16
      return c

    def fbody(r, c):
      for col in range(D // 16):
        rows[0][r, pl.ds(col * 16, 16)] = fill16
        rows[1][r, pl.ds(col * 16, 16)] = fill16
      return c

    with jax.named_scope("acc_zero"):
      lax.fori_loop(0, CHUNK, zbody, 0)
      for k in range(ROWS_PER_TILE // CHUNK):
        pltpu.sync_copy(rows[0],
                        acc.at[pl.ds(sid * ROWS_PER_TILE + k * CHUNK, CHUNK)])
      if ones_mode:
        lax.fori_loop(0, CHUNK, fbody, 0)
      plsc.subcore_barrier()

    def body(it, carry):
      for q in range(4):
        j = it * 4 + q
        b = q % 2
        pltpu.make_async_copy(edges_hbm.at[cb + j], ib[q], isem).wait()

        @pl.when(j + 1 < NCHUNK)
        def _():
          pltpu.async_copy(edges_hbm.at[cb + j + 1], ib[(q + 1) % 4], isem)

        @pl.when(j >= 2)
        def _():
          pltpu.make_async_copy(rows[b], acc.at[ib[(q + 2) % 4].at[1]],
                                ssem[b]).wait()

        if not ones_mode:
          pltpu.async_copy(g_hbm.at[ib[q].at[0]], rows[b], gsem).wait()
        pltpu.async_copy(rows[b], acc.at[ib[q].at[1]], ssem[b], add=True)
      return carry

    with jax.named_scope("edge_loop"):
      lax.fori_loop(0, NCHUNK // 4, body, 0)
      pltpu.make_async_copy(rows[0], acc.at[ib[2].at[1]], ssem[0]).wait()
      pltpu.make_async_copy(rows[1], acc.at[ib[3].at[1]], ssem[1]).wait()
    with jax.named_scope("post_barrier"):
      plsc.subcore_barrier()

    with jax.named_scope("writeout"):
      pltpu.sync_copy(acc.at[pl.ds(sid * ROWS_PER_TILE, ROWS_PER_TILE)],
                      out_hbm.at[cid, pl.ds(sid * ROWS_PER_TILE, ROWS_PER_TILE)])

  return prop


@functools.cache
def _get_propagate(ones_mode=False):
  return _make_propagate(ones_mode)


def _propagate(g, edges):
  return _get_propagate()(g, edges)


def _degree(g, edges):
  return _get_propagate(True)(g, edges)


def _prep_dinv(deg_partials):
  def body(p_ref, o_ref):
    i = pl.program_id(0)
    deg = 1.0 + p_ref[0] + p_ref[1]
    dinv = lax.rsqrt(deg)
    row = i * BM + lax.broadcasted_iota(jnp.int32, (BM, D), 0)
    o_ref[...] = jnp.where(row < N, dinv, 0.0)

  return pl.pallas_call(
      body,
      grid=(GRID,),
      in_specs=[pl.BlockSpec((NC, BM, D), lambda i: (0, i, 0))],
      out_specs=pl.BlockSpec((BM, D), lambda i: (i, 0)),
      out_shape=jax.ShapeDtypeStruct((N_PAD, D), jnp.float32),
  )(deg_partials)


def _matmul_scale(x, W, dinv_b):
  def body(x_ref, w_ref, d_ref, o_ref):
    o_ref[...] = d_ref[...] * jnp.dot(x_ref[...], w_ref[...],
                                      preferred_element_type=jnp.float32)

  return pl.pallas_call(
      body,
      grid=(GRID,),
      in_specs=[
          pl.BlockSpec((BM, D), lambda i: (i, 0)),
          pl.BlockSpec((D, D), lambda i: (0, 0)),
          pl.BlockSpec((BM, D), lambda i: (i, 0)),
      ],
      out_specs=pl.BlockSpec((BM, D), lambda i: (i, 0)),
      out_shape=jax.ShapeDtypeStruct((N_PAD, D), jnp.float32),
  )(x, W, dinv_b)


def _combine_matmul(P, g, dinv_b, bvec, gamma, beta, W):
  bn_c = float(1.0 / math.sqrt(1.0 + BN_EPS))

  def body(p_ref, g_ref, d_ref, b_ref, ga_ref, be_ref, w_ref, o_ref):
    z = d_ref[...] * (p_ref[0] + p_ref[1] + g_ref[...]) + b_ref[...]
    z = jnp.maximum(z, 0.0) * (ga_ref[...] * bn_c) + be_ref[...]
    o_ref[...] = d_ref[...] * jnp.dot(z, w_ref[...],
                                      preferred_element_type=jnp.float32)

  return pl.pallas_call(
      body,
      grid=(GRID,),
      in_specs=[
          pl.BlockSpec((NC, BM, D), lambda i: (0, i, 0)),
          pl.BlockSpec((BM, D), lambda i: (i, 0)),
          pl.BlockSpec((BM, D), lambda i: (i, 0)),
          pl.BlockSpec((1, D), lambda i: (0, 0)),
          pl.BlockSpec((1, D), lambda i: (0, 0)),
          pl.BlockSpec((1, D), lambda i: (0, 0)),
          pl.BlockSpec((D, D), lambda i: (0, 0)),
      ],
      out_specs=pl.BlockSpec((BM, D), lambda i: (i, 0)),
      out_shape=jax.ShapeDtypeStruct((N_PAD, D), jnp.float32),
  )(P, g, dinv_b, bvec, gamma, beta, W)


def _final_mean(P, g, dinv_b, bvec):
  def body(p_ref, g_ref, d_ref, b_ref, o_ref):
    i = pl.program_id(0)
    z = d_ref[...] * (p_ref[0] + p_ref[1] + g_ref[...])
    row = i * BM + lax.broadcasted_iota(jnp.int32, (BM, D), 0)
    z = jnp.where(row < N, z, 0.0)
    part = jnp.sum(z, axis=0, keepdims=True)

    @pl.when(i == 0)
    def _():
      o_ref[...] = jnp.zeros_like(o_ref)

    o_ref[...] += part

    @pl.when(i == GRID - 1)
    def _():
      o_ref[...] = o_ref[...] * (1.0 / N) + b_ref[...]

  return pl.pallas_call(
      body,
      grid=(GRID,),
      in_specs=[
          pl.BlockSpec((NC, BM, D), lambda i: (0, i, 0)),
          pl.BlockSpec((BM, D), lambda i: (i, 0)),
          pl.BlockSpec((BM, D), lambda i: (i, 0)),
          pl.BlockSpec((1, D), lambda i: (0, 0)),
      ],
      out_specs=pl.BlockSpec((1, D), lambda i: (0, 0)),
      out_shape=jax.ShapeDtypeStruct((1, D), jnp.float32),
  )(P, g, dinv_b, bvec)


@jax.jit
def _run(x, edge_index, W1, b1, W2, b2, W3, b3, W4, b4,
         gamma1, beta1, gamma2, beta2, gamma3, beta3):
  src = edge_index[0]
  dst = edge_index[1]
  pad_e = N + (jnp.arange(E_PAD - E, dtype=jnp.int32) % (N_PAD - N))
  srcp = jnp.concatenate([src, pad_e])
  dstp = jnp.concatenate([dst, pad_e])
  edges = jnp.stack([srcp.reshape(-1, CHUNK), dstp.reshape(-1, CHUNK)], axis=1)
  x_pad = jnp.pad(x, ((0, N_PAD - N), (0, 0)))


  b1r = b1.reshape(1, D)
  b2r = b2.reshape(1, D)
  b3r = b3.reshape(1, D)
  b4r = b4.reshape(1, D)
  g1r = gamma1.reshape(1, D)
  g2r = gamma2.reshape(1, D)
  g3r = gamma3.reshape(1, D)
  be1r = beta1.reshape(1, D)
  be2r = beta2.reshape(1, D)
  be3r = beta3.reshape(1, D)

  deg_p = _degree(x_pad, edges)
  dinv_b = _prep_dinv(deg_p)

  g1 = _matmul_scale(x_pad, W1, dinv_b)
  P1 = _propagate(g1, edges)
  g2 = _combine_matmul(P1, g1, dinv_b, b1r, g1r, be1r, W2)
  P2 = _propagate(g2, edges)
  g3 = _combine_matmul(P2, g2, dinv_b, b2r, g2r, be2r, W3)
  P3 = _propagate(g3, edges)
  g4 = _combine_matmul(P3, g3, dinv_b, b3r, g3r, be3r, W4)
  P4 = _propagate(g4, edges)
  return _final_mean(P4, g4, dinv_b, b4r)


def kernel(x, edge_index, W1, b1, W2, b2, W3, b3, W4, b4,
           gamma1, beta1, gamma2, beta2, gamma3, beta3):
  return _run(x, edge_index, W1, b1, W2, b2, W3, b3, W4, b4,
              gamma1, beta1, gamma2, beta2, gamma3, beta3)
